# baseline (speedup 1.0000x reference)
import jax
import jax.numpy as jnp
from jax import lax
from jax.experimental import pallas as pl
from jax.experimental.pallas import tpu as pltpu
from jax.experimental.pallas import tpu_sc as plsc

BATCH = 16384
EMBED_DIM = 32
HIDDEN = 64
N_INTER = 1000

_NC = 2
_NS = 16
_NW = _NC * _NS
_B_PER_W = BATCH // _NW
_CH = 8
_NCH = _B_PER_W // _CH
_LANES = 128
_ROWS = 128
_PCOLS = 782
_PC_PER_W = 25
_PC_ROWS = _PCOLS * 32


def _sc_gather_body(uid_hbm, pid_hbm, iid_hbm, ut_hbm, pt_hbm, it_hbm,
                    u_out, i_out, pc_out,
                    ids_v, slab0, slab1, itab_v, rows_v, pfetch_v, pstage_v,
                    sem0, sem1):
    wid = lax.axis_index("s") * _NC + lax.axis_index("c")
    base = wid * _B_PER_W

    e_lo = lax.iota(jnp.int32, 16)
    e_hi = e_lo + 16

    def gather_big(ids_hbm, tab_hbm, out_hbm):
        pltpu.sync_copy(ids_hbm.at[pl.ds(base, _B_PER_W)], ids_v)

        def fire(vv, j0, slab, sem):
            for j in range(_CH):
                tile_col = (vv[j0 + j] // _LANES) * _LANES
                pltpu.make_async_copy(
                    tab_hbm.at[:, pl.ds(tile_col, _LANES)],
                    slab.at[:, pl.ds(j * _LANES, _LANES)],
                    sem,
                ).start()

        def drain(slab, sem):
            pltpu.make_async_copy(
                tab_hbm.at[:, pl.ds(0, _CH * _LANES)], slab, sem
            ).wait()

        def extract(vv, j0, slab, cbase):
            for j in range(_CH):
                lane = lax.rem(vv[j0 + j], _LANES) + j * _LANES
                l_idx = jnp.zeros((16,), jnp.int32) + lane
                lo = plsc.load_gather(slab, [e_lo, l_idx])
                hi = plsc.load_gather(slab, [e_hi, l_idx])
                r_idx = jnp.zeros((16,), jnp.int32) + lax.rem(cbase + j, _ROWS)
                plsc.store_scatter(rows_v, [r_idx, e_lo], lo)
                plsc.store_scatter(rows_v, [r_idx, e_hi], hi)

        vv0 = ids_v[pl.ds(0, 16)]
        fire(vv0, 0, slab0, sem0)

        def body(g, _):
            vv = ids_v[pl.ds(g * 16, 16)]
            fire(vv, _CH, slab1, sem1)
            drain(slab0, sem0)
            extract(vv, 0, slab0, g * 16)

            @pl.when(g < _NCH // 2 - 1)
            def _fire_next():
                vvn = ids_v[pl.ds(g * 16 + 16, 16)]
                fire(vvn, 0, slab0, sem0)

            drain(slab1, sem1)
            extract(vv, _CH, slab1, g * 16 + _CH)

            @pl.when(lax.rem(g, 8) == 7)
            def _flush():
                pltpu.sync_copy(
                    rows_v,
                    out_hbm.at[pl.ds(base + (g // 8) * _ROWS, _ROWS), :])

            return _

        lax.fori_loop(0, _NCH // 2, body, 0)

    gather_big(uid_hbm, ut_hbm, u_out)

    def pcol(c, _):
        gcol = wid * _PC_PER_W + c

        @pl.when(gcol < _PCOLS)
        def _do():
            off = pl.multiple_of(gcol * _LANES, _LANES)
            pltpu.sync_copy(pt_hbm.at[:, pl.ds(off, _LANES)], pfetch_v)

            def lgrp(gl, _unused):
                vbase = gl * 16
                for j in range(16):
                    l = vbase + j
                    l_idx = jnp.zeros((16,), jnp.int32) + l
                    lo = plsc.load_gather(pfetch_v, [e_lo, l_idx])
                    hi = plsc.load_gather(pfetch_v, [e_hi, l_idx])
                    r_idx = jnp.zeros((16,), jnp.int32) + (l // 4)
                    w0 = lax.rem(l, 4) * 32
                    plsc.store_scatter(pstage_v, [r_idx, e_lo + w0], lo)
                    plsc.store_scatter(pstage_v, [r_idx, e_hi + w0], hi)
                return _unused

            lax.fori_loop(0, _LANES // 16, lgrp, 0)
            rowoff = pl.multiple_of(gcol * 32, 8)
            pltpu.sync_copy(pstage_v, pc_out.at[pl.ds(rowoff, 32), :])

        return _

    lax.fori_loop(0, _PC_PER_W, pcol, 0)

    pltpu.sync_copy(it_hbm, itab_v)
    pltpu.sync_copy(iid_hbm.at[pl.ds(base, _B_PER_W)], ids_v)

    def ichunk(g, _):
        vv = ids_v[pl.ds(g * 16, 16)]
        for j in range(16):
            l_idx = jnp.zeros((16,), jnp.int32) + vv[j]
            lo = plsc.load_gather(itab_v, [e_lo, l_idx])
            hi = plsc.load_gather(itab_v, [e_hi, l_idx])
            r_idx = jnp.zeros((16,), jnp.int32) + lax.rem(g * 16 + j, _ROWS)
            plsc.store_scatter(rows_v, [r_idx, e_lo], lo)
            plsc.store_scatter(rows_v, [r_idx, e_hi], hi)

        @pl.when(lax.rem(g, 8) == 7)
        def _flush():
            pltpu.sync_copy(
                rows_v, i_out.at[pl.ds(base + (g // 8) * _ROWS, _ROWS), :])

        return _

    lax.fori_loop(0, _B_PER_W // 16, ichunk, 0)


@jax.jit
def _sc_gather(user_ids, product_ids, interaction_ids, ut_t, pt_t, it_t):
    mesh = plsc.VectorSubcoreMesh(core_axis_name="c", subcore_axis_name="s")
    f = pl.kernel(
        _sc_gather_body,
        out_type=[jax.ShapeDtypeStruct((BATCH, EMBED_DIM), jnp.float32)] * 2
        + [jax.ShapeDtypeStruct((_PC_ROWS, 128), jnp.float32)],
        mesh=mesh,
        scratch_types=[
            pltpu.VMEM((_B_PER_W,), jnp.int32),
            pltpu.VMEM((EMBED_DIM, _CH * _LANES), jnp.float32),
            pltpu.VMEM((EMBED_DIM, _CH * _LANES), jnp.float32),
            pltpu.VMEM((EMBED_DIM, N_INTER), jnp.float32),
            pltpu.VMEM((_ROWS, EMBED_DIM), jnp.float32),
            pltpu.VMEM((EMBED_DIM, _LANES), jnp.float32),
            pltpu.VMEM((32, _LANES), jnp.float32),
            pltpu.SemaphoreType.DMA,
            pltpu.SemaphoreType.DMA,
        ],
        compiler_params=pltpu.CompilerParams(
            use_tc_tiling_on_sc=True, needs_layout_passes=False),
    )
    return f(user_ids, product_ids, interaction_ids, ut_t, pt_t, it_t)


def _sc_pgather_body(pid_hbm, pc_hbm, p_out,
                     ids_v, idx_v, pbuf_v, prow_v, sem):
    wid = lax.axis_index("s") * _NC + lax.axis_index("c")
    base = wid * _B_PER_W
    e_lo = lax.iota(jnp.int32, 16)
    e_hi = e_lo + 16

    pltpu.sync_copy(pid_hbm.at[pl.ds(base, _B_PER_W)], ids_v)

    def mkidx(g, _):
        vv = ids_v[pl.ds(g * 16, 16)]
        idx_v[pl.ds(g * 16, 16)] = vv // 4
        return _

    lax.fori_loop(0, _B_PER_W // 16, mkidx, 0)

    def rnd(r, _):
        pltpu.async_copy(
            pc_hbm.at[idx_v.at[pl.ds(r * _ROWS, _ROWS)]], pbuf_v, sem
        ).wait()

        def grp(gl, _unused):
            vv = ids_v[pl.ds(r * _ROWS + gl * 16, 16)]
            for j in range(16):
                w0 = lax.rem(vv[j], 4) * 32
                r_idx = jnp.zeros((16,), jnp.int32) + (gl * 16 + j)
                lo = plsc.load_gather(pbuf_v, [r_idx, e_lo + w0])
                hi = plsc.load_gather(pbuf_v, [r_idx, e_hi + w0])
                plsc.store_scatter(prow_v, [r_idx, e_lo], lo)
                plsc.store_scatter(prow_v, [r_idx, e_hi], hi)
            return _unused

        lax.fori_loop(0, _ROWS // 16, grp, 0)
        pltpu.sync_copy(prow_v, p_out.at[pl.ds(base + r * _ROWS, _ROWS), :])
        return _

    lax.fori_loop(0, _B_PER_W // _ROWS, rnd, 0)


@jax.jit
def _sc_pgather(product_ids, pc):
    mesh = plsc.VectorSubcoreMesh(core_axis_name="c", subcore_axis_name="s")
    f = pl.kernel(
        _sc_pgather_body,
        out_type=jax.ShapeDtypeStruct((BATCH, EMBED_DIM), jnp.float32),
        mesh=mesh,
        scratch_types=[
            pltpu.VMEM((_B_PER_W,), jnp.int32),
            pltpu.VMEM((_B_PER_W,), jnp.int32),
            pltpu.VMEM((_ROWS, 128), jnp.float32),
            pltpu.VMEM((_ROWS, EMBED_DIM), jnp.float32),
            pltpu.SemaphoreType.DMA,
        ],
        compiler_params=pltpu.CompilerParams(
            use_tc_tiling_on_sc=False, needs_layout_passes=False),
    )
    return f(product_ids, pc)


def _mlp_body(u_ref, p_ref, i_ref, w1_ref, b1_ref, w2_ref, b2_ref, o_ref):
    h = jnp.dot(u_ref[...], w1_ref[0:EMBED_DIM, :],
                preferred_element_type=jnp.float32)
    h = h + jnp.dot(p_ref[...], w1_ref[EMBED_DIM:2 * EMBED_DIM, :],
                    preferred_element_type=jnp.float32)
    h = h + jnp.dot(i_ref[...], w1_ref[2 * EMBED_DIM:3 * EMBED_DIM, :],
                    preferred_element_type=jnp.float32)
    h = jnp.maximum(h + b1_ref[...], 0.0)
    o_ref[...] = jnp.dot(h, w2_ref[...],
                         preferred_element_type=jnp.float32) + b2_ref[...]


_MLP_BLK = 4096


@jax.jit
def _mlp(u, p, i, W1, b1, W2, b2):
    grid = (BATCH // _MLP_BLK,)
    return pl.pallas_call(
        _mlp_body,
        grid=grid,
        in_specs=[
            pl.BlockSpec((_MLP_BLK, EMBED_DIM), lambda g: (g, 0)),
            pl.BlockSpec((_MLP_BLK, EMBED_DIM), lambda g: (g, 0)),
            pl.BlockSpec((_MLP_BLK, EMBED_DIM), lambda g: (g, 0)),
            pl.BlockSpec((3 * EMBED_DIM, HIDDEN), lambda g: (0, 0)),
            pl.BlockSpec((1, HIDDEN), lambda g: (0, 0)),
            pl.BlockSpec((HIDDEN, 1), lambda g: (0, 0)),
            pl.BlockSpec((1, 1), lambda g: (0, 0)),
        ],
        out_specs=pl.BlockSpec((_MLP_BLK, 1), lambda g: (g, 0)),
        out_shape=jax.ShapeDtypeStruct((BATCH, 1), jnp.float32),
    )(u, p, i, W1, b1, W2, b2)


def kernel(user_ids, product_ids, interaction_ids, user_table, product_table,
           interaction_table, W1, b1, W2, b2):
    uids = user_ids.astype(jnp.int32)
    pids = product_ids.astype(jnp.int32)
    iids = interaction_ids.astype(jnp.int32)
    u, i, pc = _sc_gather(uids, pids, iids, user_table.T,
                          product_table.T, interaction_table.T)
    p = _sc_pgather(pids, pc)
    return _mlp(u, p, i, W1, b1.reshape(1, HIDDEN), W2, b2.reshape(1, 1))


# speedup vs baseline: 1.0131x; 1.0131x over previous
import jax
import jax.numpy as jnp
from jax import lax
from jax.experimental import pallas as pl
from jax.experimental.pallas import tpu as pltpu
from jax.experimental.pallas import tpu_sc as plsc

BATCH = 16384
EMBED_DIM = 32
HIDDEN = 64
N_INTER = 1000

_NC = 2
_NS = 16
_NW = _NC * _NS
_B_PER_W = BATCH // _NW
_CH = 4
_LANES = 128
_ROWS = 128
_PCOLS = 782
_ICOLS = 8
_TCOLS = _PCOLS + _ICOLS
_PC_PER_W = 25
_PC_ROWS = _TCOLS * 32
_IOFF = _PCOLS * 32


def _sc_gather_body(uid_hbm, pid_hbm, iid_hbm, ut_hbm, pt_hbm, it_hbm,
                    u_out, pc_out,
                    ids_v, slab0, slab1, slab2, slab3, rows_v,
                    pfetch_v, pstage_v, sem0, sem1, sem2, sem3):
    wid = lax.axis_index("s") * _NC + lax.axis_index("c")
    base = wid * _B_PER_W

    e_lo = lax.iota(jnp.int32, 16)
    e_hi = e_lo + 16

    slabs = (slab0, slab1, slab2, slab3)
    sems = (sem0, sem1, sem2, sem3)

    pltpu.sync_copy(uid_hbm.at[pl.ds(base, _B_PER_W)], ids_v)

    def fire(vv, j0, q):
        for j in range(_CH):
            tile_col = (vv[j0 + j] // _LANES) * _LANES
            pltpu.make_async_copy(
                ut_hbm.at[:, pl.ds(tile_col, _LANES)],
                slabs[q].at[:, pl.ds(j * _LANES, _LANES)],
                sems[q],
            ).start()

    def drain(q):
        pltpu.make_async_copy(
            ut_hbm.at[:, pl.ds(0, _CH * _LANES)], slabs[q], sems[q]
        ).wait()

    def extract(vv, j0, q, cbase):
        for j in range(_CH):
            lane = lax.rem(vv[j0 + j], _LANES) + j * _LANES
            l_idx = jnp.zeros((16,), jnp.int32) + lane
            lo = plsc.load_gather(slabs[q], [e_lo, l_idx])
            hi = plsc.load_gather(slabs[q], [e_hi, l_idx])
            r_idx = jnp.zeros((16,), jnp.int32) + lax.rem(cbase + j, _ROWS)
            plsc.store_scatter(rows_v, [r_idx, e_lo], lo)
            plsc.store_scatter(rows_v, [r_idx, e_hi], hi)

    vv0 = ids_v[pl.ds(0, 16)]
    fire(vv0, 0, 0)
    fire(vv0, _CH, 1)
    fire(vv0, 2 * _CH, 2)

    def body(g, _):
        vv = ids_v[pl.ds(g * 16, 16)]
        drain(0)
        extract(vv, 0, 0, g * 16)
        fire(vv, 3 * _CH, 3)

        gn = jnp.minimum(g + 1, _B_PER_W // 16 - 1)
        vvn = ids_v[pl.ds(gn * 16, 16)]
        not_last = g < _B_PER_W // 16 - 1

        drain(1)
        extract(vv, _CH, 1, g * 16 + _CH)

        @pl.when(not_last)
        def _f0():
            fire(vvn, 0, 0)

        drain(2)
        extract(vv, 2 * _CH, 2, g * 16 + 2 * _CH)

        @pl.when(not_last)
        def _f1():
            fire(vvn, _CH, 1)

        drain(3)
        extract(vv, 3 * _CH, 3, g * 16 + 3 * _CH)

        @pl.when(not_last)
        def _f2():
            fire(vvn, 2 * _CH, 2)

        @pl.when(lax.rem(g, 8) == 7)
        def _flush():
            pltpu.sync_copy(
                rows_v, u_out.at[pl.ds(base + (g // 8) * _ROWS, _ROWS), :])

        return _

    lax.fori_loop(0, _B_PER_W // 16, body, 0)

    def pcol(c, _):
        gcol = wid * _PC_PER_W + c

        def repack():
            def lgrp(gl, _unused):
                for j in range(16):
                    l = gl * 16 + j
                    l_idx = jnp.zeros((16,), jnp.int32) + l
                    lo = plsc.load_gather(pfetch_v, [e_lo, l_idx])
                    hi = plsc.load_gather(pfetch_v, [e_hi, l_idx])
                    r_idx = jnp.zeros((16,), jnp.int32) + (l // 4)
                    w0 = lax.rem(l, 4) * 32
                    plsc.store_scatter(pstage_v, [r_idx, e_lo + w0], lo)
                    plsc.store_scatter(pstage_v, [r_idx, e_hi + w0], hi)
                return _unused

            lax.fori_loop(0, _LANES // 16, lgrp, 0)
            rowoff = pl.multiple_of(gcol * 32, 8)
            pltpu.sync_copy(pstage_v, pc_out.at[pl.ds(rowoff, 32), :])

        @pl.when(gcol < _PCOLS)
        def _prod():
            off = pl.multiple_of(gcol * _LANES, _LANES)
            pltpu.sync_copy(pt_hbm.at[:, pl.ds(off, _LANES)], pfetch_v)
            repack()

        @pl.when(jnp.logical_and(gcol >= _PCOLS, gcol < _TCOLS))
        def _inter():
            ioff = pl.multiple_of((gcol - _PCOLS) * _LANES, _LANES)
            pltpu.sync_copy(it_hbm.at[:, pl.ds(ioff, _LANES)], pfetch_v)
            repack()

        return _

    lax.fori_loop(0, _PC_PER_W, pcol, 0)


@jax.jit
def _sc_gather(user_ids, product_ids, interaction_ids, ut_t, pt_t, it_t):
    mesh = plsc.VectorSubcoreMesh(core_axis_name="c", subcore_axis_name="s")
    f = pl.kernel(
        _sc_gather_body,
        out_type=[jax.ShapeDtypeStruct((BATCH, EMBED_DIM), jnp.float32),
                  jax.ShapeDtypeStruct((_PC_ROWS, 128), jnp.float32)],
        mesh=mesh,
        scratch_types=[
            pltpu.VMEM((_B_PER_W,), jnp.int32),
            pltpu.VMEM((EMBED_DIM, _CH * _LANES), jnp.float32),
            pltpu.VMEM((EMBED_DIM, _CH * _LANES), jnp.float32),
            pltpu.VMEM((EMBED_DIM, _CH * _LANES), jnp.float32),
            pltpu.VMEM((EMBED_DIM, _CH * _LANES), jnp.float32),
            pltpu.VMEM((_ROWS, EMBED_DIM), jnp.float32),
            pltpu.VMEM((EMBED_DIM, _LANES), jnp.float32),
            pltpu.VMEM((32, _LANES), jnp.float32),
            pltpu.SemaphoreType.DMA,
            pltpu.SemaphoreType.DMA,
            pltpu.SemaphoreType.DMA,
            pltpu.SemaphoreType.DMA,
        ],
        compiler_params=pltpu.CompilerParams(
            use_tc_tiling_on_sc=True, needs_layout_passes=False),
    )
    return f(user_ids, product_ids, interaction_ids, ut_t, pt_t, it_t)


def _sc_pgather_body(pid_hbm, iid_hbm, pc_hbm, p_out, i_out,
                     ids_v, idx_v, pbuf_v, prow_v, sem):
    wid = lax.axis_index("s") * _NC + lax.axis_index("c")
    base = wid * _B_PER_W
    e_lo = lax.iota(jnp.int32, 16)
    e_hi = e_lo + 16

    def run(ids_hbm, out_hbm, roff):
        pltpu.sync_copy(ids_hbm.at[pl.ds(base, _B_PER_W)], ids_v)

        def mkidx(g, _):
            vv = ids_v[pl.ds(g * 16, 16)]
            idx_v[pl.ds(g * 16, 16)] = vv // 4 + roff
            return _

        lax.fori_loop(0, _B_PER_W // 16, mkidx, 0)

        def rnd(r, _):
            pltpu.async_copy(
                pc_hbm.at[idx_v.at[pl.ds(r * _ROWS, _ROWS)]], pbuf_v, sem
            ).wait()

            def grp(gl, _unused):
                vv = ids_v[pl.ds(r * _ROWS + gl * 16, 16)]
                for j in range(16):
                    w0 = lax.rem(vv[j], 4) * 32
                    r_idx = jnp.zeros((16,), jnp.int32) + (gl * 16 + j)
                    lo = plsc.load_gather(pbuf_v, [r_idx, e_lo + w0])
                    hi = plsc.load_gather(pbuf_v, [r_idx, e_hi + w0])
                    plsc.store_scatter(prow_v, [r_idx, e_lo], lo)
                    plsc.store_scatter(prow_v, [r_idx, e_hi], hi)
                return _unused

            lax.fori_loop(0, _ROWS // 16, grp, 0)
            pltpu.sync_copy(prow_v,
                            out_hbm.at[pl.ds(base + r * _ROWS, _ROWS), :])
            return _

        lax.fori_loop(0, _B_PER_W // _ROWS, rnd, 0)

    run(pid_hbm, p_out, 0)
    run(iid_hbm, i_out, _IOFF)


@jax.jit
def _sc_pgather(product_ids, interaction_ids, pc):
    mesh = plsc.VectorSubcoreMesh(core_axis_name="c", subcore_axis_name="s")
    f = pl.kernel(
        _sc_pgather_body,
        out_type=[jax.ShapeDtypeStruct((BATCH, EMBED_DIM), jnp.float32)] * 2,
        mesh=mesh,
        scratch_types=[
            pltpu.VMEM((_B_PER_W,), jnp.int32),
            pltpu.VMEM((_B_PER_W,), jnp.int32),
            pltpu.VMEM((_ROWS, 128), jnp.float32),
            pltpu.VMEM((_ROWS, EMBED_DIM), jnp.float32),
            pltpu.SemaphoreType.DMA,
        ],
        compiler_params=pltpu.CompilerParams(
            use_tc_tiling_on_sc=False, needs_layout_passes=False),
    )
    return f(product_ids, interaction_ids, pc)


def _mlp_body(u_ref, p_ref, i_ref, w1_ref, b1_ref, w2_ref, b2_ref, o_ref):
    h = jnp.dot(u_ref[...], w1_ref[0:EMBED_DIM, :],
                preferred_element_type=jnp.float32)
    h = h + jnp.dot(p_ref[...], w1_ref[EMBED_DIM:2 * EMBED_DIM, :],
                    preferred_element_type=jnp.float32)
    h = h + jnp.dot(i_ref[...], w1_ref[2 * EMBED_DIM:3 * EMBED_DIM, :],
                    preferred_element_type=jnp.float32)
    h = jnp.maximum(h + b1_ref[...], 0.0)
    o_ref[...] = jnp.dot(h, w2_ref[...],
                         preferred_element_type=jnp.float32) + b2_ref[...]


_MLP_BLK = 4096


@jax.jit
def _mlp(u, p, i, W1, b1, W2, b2):
    grid = (BATCH // _MLP_BLK,)
    return pl.pallas_call(
        _mlp_body,
        grid=grid,
        in_specs=[
            pl.BlockSpec((_MLP_BLK, EMBED_DIM), lambda g: (g, 0)),
            pl.BlockSpec((_MLP_BLK, EMBED_DIM), lambda g: (g, 0)),
            pl.BlockSpec((_MLP_BLK, EMBED_DIM), lambda g: (g, 0)),
            pl.BlockSpec((3 * EMBED_DIM, HIDDEN), lambda g: (0, 0)),
            pl.BlockSpec((1, HIDDEN), lambda g: (0, 0)),
            pl.BlockSpec((HIDDEN, 1), lambda g: (0, 0)),
            pl.BlockSpec((1, 1), lambda g: (0, 0)),
        ],
        out_specs=pl.BlockSpec((_MLP_BLK, 1), lambda g: (g, 0)),
        out_shape=jax.ShapeDtypeStruct((BATCH, 1), jnp.float32),
    )(u, p, i, W1, b1, W2, b2)


def kernel(user_ids, product_ids, interaction_ids, user_table, product_table,
           interaction_table, W1, b1, W2, b2):
    uids = user_ids.astype(jnp.int32)
    pids = product_ids.astype(jnp.int32)
    iids = interaction_ids.astype(jnp.int32)
    u, pc = _sc_gather(uids, pids, iids, user_table.T,
                       product_table.T, interaction_table.T)
    p, i = _sc_pgather(pids, iids, pc)
    return _mlp(u, p, i, W1, b1.reshape(1, HIDDEN), W2, b2.reshape(1, 1))


# speedup vs baseline: 1.0236x; 1.0104x over previous
import jax
import jax.numpy as jnp
from jax import lax
from jax.experimental import pallas as pl
from jax.experimental.pallas import tpu as pltpu
from jax.experimental.pallas import tpu_sc as plsc

BATCH = 16384
EMBED_DIM = 32
HIDDEN = 64
N_INTER = 1000

_NC = 2
_NS = 16
_NW = _NC * _NS
_B_PER_W = BATCH // _NW
_CH = 4
_LANES = 128
_ROWS = 128
_PCOLS = 782
_ICOLS = 8
_TCOLS = _PCOLS + _ICOLS
_PC_PER_W = 25
_PC_ROWS = _TCOLS * 32
_IOFF = _PCOLS * 32


def _sc_gather_body(uid_hbm, pid_hbm, iid_hbm, ut_hbm, pt_hbm, it_hbm,
                    u_out, pc_out,
                    ids_v, slab0, slab1, slab2, slab3, rows_v,
                    pfetch_v, pstage_v, sem0, sem1, sem2, sem3):
    wid = lax.axis_index("s") * _NC + lax.axis_index("c")
    base = wid * _B_PER_W

    e_lo = lax.iota(jnp.int32, 16)
    e_hi = e_lo + 16

    slabs = (slab0, slab1, slab2, slab3)
    sems = (sem0, sem1, sem2, sem3)

    pltpu.sync_copy(uid_hbm.at[pl.ds(base, _B_PER_W)], ids_v)

    def fire(vv, j0, q):
        for j in range(_CH):
            tile_col = (vv[j0 + j] // _LANES) * _LANES
            pltpu.make_async_copy(
                ut_hbm.at[:, pl.ds(tile_col, _LANES)],
                slabs[q].at[:, pl.ds(j * _LANES, _LANES)],
                sems[q],
            ).start()

    def drain(q):
        pltpu.make_async_copy(
            ut_hbm.at[:, pl.ds(0, _CH * _LANES)], slabs[q], sems[q]
        ).wait()

    def extract(vv, j0, q, cbase):
        for j in range(_CH):
            lane = lax.rem(vv[j0 + j], _LANES) + j * _LANES
            l_idx = jnp.zeros((16,), jnp.int32) + lane
            lo = plsc.load_gather(slabs[q], [e_lo, l_idx])
            hi = plsc.load_gather(slabs[q], [e_hi, l_idx])
            r_idx = jnp.zeros((16,), jnp.int32) + lax.rem(cbase + j, _ROWS)
            plsc.store_scatter(rows_v, [r_idx, e_lo], lo)
            plsc.store_scatter(rows_v, [r_idx, e_hi], hi)

    vv0 = ids_v[pl.ds(0, 16)]
    fire(vv0, 0, 0)
    fire(vv0, _CH, 1)
    fire(vv0, 2 * _CH, 2)

    def body(g, _):
        vv = ids_v[pl.ds(g * 16, 16)]
        drain(0)
        extract(vv, 0, 0, g * 16)
        fire(vv, 3 * _CH, 3)

        gn = jnp.minimum(g + 1, _B_PER_W // 16 - 1)
        vvn = ids_v[pl.ds(gn * 16, 16)]
        not_last = g < _B_PER_W // 16 - 1

        drain(1)
        extract(vv, _CH, 1, g * 16 + _CH)

        @pl.when(not_last)
        def _f0():
            fire(vvn, 0, 0)

        drain(2)
        extract(vv, 2 * _CH, 2, g * 16 + 2 * _CH)

        @pl.when(not_last)
        def _f1():
            fire(vvn, _CH, 1)

        drain(3)
        extract(vv, 3 * _CH, 3, g * 16 + 3 * _CH)

        @pl.when(not_last)
        def _f2():
            fire(vvn, 2 * _CH, 2)

        @pl.when(lax.rem(g, 8) == 7)
        def _flush():
            pltpu.sync_copy(
                rows_v, u_out.at[pl.ds(base + (g // 8) * _ROWS, _ROWS), :])

        return _

    lax.fori_loop(0, _B_PER_W // 16, body, 0)

    def pcol(c, _):
        gcol = wid * _PC_PER_W + c

        def repack():
            def lgrp(gl, _unused):
                for j in range(16):
                    l = gl * 16 + j
                    l_idx = jnp.zeros((16,), jnp.int32) + l
                    lo = plsc.load_gather(pfetch_v, [e_lo, l_idx])
                    hi = plsc.load_gather(pfetch_v, [e_hi, l_idx])
                    r_idx = jnp.zeros((16,), jnp.int32) + (l // 4)
                    w0 = lax.rem(l, 4) * 32
                    plsc.store_scatter(pstage_v, [r_idx, e_lo + w0], lo)
                    plsc.store_scatter(pstage_v, [r_idx, e_hi + w0], hi)
                return _unused

            lax.fori_loop(0, _LANES // 16, lgrp, 0)
            rowoff = pl.multiple_of(gcol * 32, 8)
            pltpu.sync_copy(pstage_v, pc_out.at[pl.ds(rowoff, 32), :])

        @pl.when(gcol < _PCOLS)
        def _prod():
            off = pl.multiple_of(gcol * _LANES, _LANES)
            pltpu.sync_copy(pt_hbm.at[:, pl.ds(off, _LANES)], pfetch_v)
            repack()

        @pl.when(jnp.logical_and(gcol >= _PCOLS, gcol < _TCOLS))
        def _inter():
            ioff = pl.multiple_of((gcol - _PCOLS) * _LANES, _LANES)
            pltpu.sync_copy(it_hbm.at[:, pl.ds(ioff, _LANES)], pfetch_v)
            repack()

        return _

    lax.fori_loop(0, _PC_PER_W, pcol, 0)


@jax.jit
def _sc_gather(user_ids, product_ids, interaction_ids, ut_t, pt_t, it_t):
    mesh = plsc.VectorSubcoreMesh(core_axis_name="c", subcore_axis_name="s")
    f = pl.kernel(
        _sc_gather_body,
        out_type=[jax.ShapeDtypeStruct((BATCH, EMBED_DIM), jnp.float32),
                  jax.ShapeDtypeStruct((_PC_ROWS, 128), jnp.float32)],
        mesh=mesh,
        scratch_types=[
            pltpu.VMEM((_B_PER_W,), jnp.int32),
            pltpu.VMEM((EMBED_DIM, _CH * _LANES), jnp.float32),
            pltpu.VMEM((EMBED_DIM, _CH * _LANES), jnp.float32),
            pltpu.VMEM((EMBED_DIM, _CH * _LANES), jnp.float32),
            pltpu.VMEM((EMBED_DIM, _CH * _LANES), jnp.float32),
            pltpu.VMEM((_ROWS, EMBED_DIM), jnp.float32),
            pltpu.VMEM((EMBED_DIM, _LANES), jnp.float32),
            pltpu.VMEM((32, _LANES), jnp.float32),
            pltpu.SemaphoreType.DMA,
            pltpu.SemaphoreType.DMA,
            pltpu.SemaphoreType.DMA,
            pltpu.SemaphoreType.DMA,
        ],
        compiler_params=pltpu.CompilerParams(
            use_tc_tiling_on_sc=True, needs_layout_passes=False),
    )
    return f(user_ids, product_ids, interaction_ids, ut_t, pt_t, it_t)


def _sc_pgather_body(pid_hbm, iid_hbm, pc_hbm, p_out, i_out,
                     ids_v, idx_v, pbuf_v, prow_v, sem):
    wid = lax.axis_index("s") * _NC + lax.axis_index("c")
    base = wid * _B_PER_W
    e_lo = lax.iota(jnp.int32, 16)
    e_hi = e_lo + 16

    def run(ids_hbm, out_hbm, roff):
        pltpu.sync_copy(ids_hbm.at[pl.ds(base, _B_PER_W)], ids_v)

        def mkidx(g, _):
            vv = ids_v[pl.ds(g * 16, 16)]
            idx_v[pl.ds(g * 16, 16)] = vv // 4 + roff
            return _

        lax.fori_loop(0, _B_PER_W // 16, mkidx, 0)

        def rnd(r, _):
            pltpu.async_copy(
                pc_hbm.at[idx_v.at[pl.ds(r * _ROWS, _ROWS)]], pbuf_v, sem
            ).wait()

            def grp(gl, _unused):
                vv = ids_v[pl.ds(r * _ROWS + gl * 16, 16)]
                for j in range(16):
                    w0 = lax.rem(vv[j], 4) * 32
                    r_idx = jnp.zeros((16,), jnp.int32) + (gl * 16 + j)
                    lo = plsc.load_gather(pbuf_v, [r_idx, e_lo + w0])
                    hi = plsc.load_gather(pbuf_v, [r_idx, e_hi + w0])
                    plsc.store_scatter(prow_v, [r_idx, e_lo], lo)
                    plsc.store_scatter(prow_v, [r_idx, e_hi], hi)
                return _unused

            lax.fori_loop(0, _ROWS // 16, grp, 0)
            pltpu.sync_copy(prow_v,
                            out_hbm.at[pl.ds(base + r * _ROWS, _ROWS), :])
            return _

        lax.fori_loop(0, _B_PER_W // _ROWS, rnd, 0)

    run(pid_hbm, p_out, 0)
    run(iid_hbm, i_out, _IOFF)


@jax.jit
def _sc_pgather(product_ids, interaction_ids, pc):
    mesh = plsc.VectorSubcoreMesh(core_axis_name="c", subcore_axis_name="s")
    f = pl.kernel(
        _sc_pgather_body,
        out_type=[jax.ShapeDtypeStruct((BATCH, EMBED_DIM), jnp.float32)] * 2,
        mesh=mesh,
        scratch_types=[
            pltpu.VMEM((_B_PER_W,), jnp.int32),
            pltpu.VMEM((_B_PER_W,), jnp.int32),
            pltpu.VMEM((_ROWS, 128), jnp.float32),
            pltpu.VMEM((_ROWS, EMBED_DIM), jnp.float32),
            pltpu.SemaphoreType.DMA,
        ],
        compiler_params=pltpu.CompilerParams(
            use_tc_tiling_on_sc=True, needs_layout_passes=False),
    )
    return f(product_ids, interaction_ids, pc)


def _mlp_body(u_ref, p_ref, i_ref, w1_ref, b1_ref, w2_ref, b2_ref, o_ref):
    h = jnp.dot(u_ref[...], w1_ref[0:EMBED_DIM, :],
                preferred_element_type=jnp.float32)
    h = h + jnp.dot(p_ref[...], w1_ref[EMBED_DIM:2 * EMBED_DIM, :],
                    preferred_element_type=jnp.float32)
    h = h + jnp.dot(i_ref[...], w1_ref[2 * EMBED_DIM:3 * EMBED_DIM, :],
                    preferred_element_type=jnp.float32)
    h = jnp.maximum(h + b1_ref[...], 0.0)
    o_ref[...] = jnp.dot(h, w2_ref[...],
                         preferred_element_type=jnp.float32) + b2_ref[...]


_MLP_BLK = 4096


@jax.jit
def _mlp(u, p, i, W1, b1, W2, b2):
    grid = (BATCH // _MLP_BLK,)
    return pl.pallas_call(
        _mlp_body,
        grid=grid,
        in_specs=[
            pl.BlockSpec((_MLP_BLK, EMBED_DIM), lambda g: (g, 0)),
            pl.BlockSpec((_MLP_BLK, EMBED_DIM), lambda g: (g, 0)),
            pl.BlockSpec((_MLP_BLK, EMBED_DIM), lambda g: (g, 0)),
            pl.BlockSpec((3 * EMBED_DIM, HIDDEN), lambda g: (0, 0)),
            pl.BlockSpec((1, HIDDEN), lambda g: (0, 0)),
            pl.BlockSpec((HIDDEN, 1), lambda g: (0, 0)),
            pl.BlockSpec((1, 1), lambda g: (0, 0)),
        ],
        out_specs=pl.BlockSpec((_MLP_BLK, 1), lambda g: (g, 0)),
        out_shape=jax.ShapeDtypeStruct((BATCH, 1), jnp.float32),
    )(u, p, i, W1, b1, W2, b2)


def kernel(user_ids, product_ids, interaction_ids, user_table, product_table,
           interaction_table, W1, b1, W2, b2):
    uids = user_ids.astype(jnp.int32)
    pids = product_ids.astype(jnp.int32)
    iids = interaction_ids.astype(jnp.int32)
    u, pc = _sc_gather(uids, pids, iids, user_table.T,
                       product_table.T, interaction_table.T)
    p, i = _sc_pgather(pids, iids, pc)
    return _mlp(u, p, i, W1, b1.reshape(1, HIDDEN), W2, b2.reshape(1, 1))
